# baseline (speedup 1.0000x reference)
import functools

import jax
import jax.numpy as jnp
from jax import lax
from jax.experimental import pallas as pl
from jax.experimental.pallas import tpu as pltpu
from jax.experimental.pallas import tpu_sc as plsc

N = 10000
D = 128
E = 320000
G = 64

CHUNK = 128
N_PAD = 10240
E_CHUNKS = 2560
EC_TILE = E_CHUNKS // 32
STRIPE = N_PAD // 16
P_TILES = 20
BCH_TILE = (N_PAD // CHUNK) // P_TILES
DUMMY = N_PAD - 8
G_PAD = 128

_mesh = plsc.VectorSubcoreMesh(core_axis_name="c", subcore_axis_name="s")


@functools.partial(
    pl.kernel,
    mesh=_mesh,
    out_type=jax.ShapeDtypeStruct((2, N_PAD, D), jnp.float32),
    scratch_types=(
        pltpu.VMEM((EC_TILE, CHUNK), jnp.int32),
        pltpu.VMEM((EC_TILE, CHUNK), jnp.int32),
        pltpu.VMEM((CHUNK, D), jnp.float32),
        pltpu.VMEM_SHARED((N_PAD, D), jnp.float32),
        pltpu.SemaphoreType.DMA,
    ),
)
def _agg(y_hbm, src_hbm, dst_hbm, zeros_hbm, out_hbm,
         srcv, dstv, rows, acc, sem):
    c = lax.axis_index("c")
    s = lax.axis_index("s")
    w = c * 16 + s
    pltpu.sync_copy(zeros_hbm.at[pl.ds(s * STRIPE, STRIPE)],
                    acc.at[pl.ds(s * STRIPE, STRIPE)])
    pltpu.sync_copy(src_hbm.at[w], srcv)
    pltpu.sync_copy(dst_hbm.at[w], dstv)
    plsc.subcore_barrier()

    def body(j, carry):
        pltpu.async_copy(y_hbm.at[srcv.at[j]], rows, sem).wait()
        pltpu.sync_copy(rows, acc.at[dstv.at[j]], add=True)
        return carry

    lax.fori_loop(0, EC_TILE, body, 0)
    plsc.subcore_barrier()
    pltpu.sync_copy(acc.at[pl.ds(s * STRIPE, STRIPE)],
                    out_hbm.at[c].at[pl.ds(s * STRIPE, STRIPE)])


@functools.partial(
    pl.kernel,
    mesh=_mesh,
    out_type=jax.ShapeDtypeStruct((2, G_PAD, D), jnp.float32),
    scratch_types=(
        pltpu.VMEM((BCH_TILE, CHUNK), jnp.int32),
        pltpu.VMEM((CHUNK, D), jnp.float32),
        pltpu.VMEM_SHARED((G_PAD, D), jnp.float32),
    ),
)
def _pool(h_hbm, batch_hbm, zeros_hbm, out_hbm, batv, rows, acc):
    c = lax.axis_index("c")
    s = lax.axis_index("s")
    w = c * 16 + s

    @pl.when(s == 0)
    def _():
        pltpu.sync_copy(zeros_hbm.at[pl.ds(0, G_PAD)], acc)

    @pl.when(w < P_TILES)
    def _():
        pltpu.sync_copy(batch_hbm.at[jnp.minimum(w, P_TILES - 1)], batv)

    plsc.subcore_barrier()

    @pl.when(w < P_TILES)
    def _():
        for j in range(BCH_TILE):
            pltpu.sync_copy(
                h_hbm.at[pl.ds((w * BCH_TILE + j) * CHUNK, CHUNK)], rows)
            pltpu.sync_copy(rows, acc.at[batv.at[j]], add=True)

    plsc.subcore_barrier()

    @pl.when(s == 0)
    def _():
        pltpu.sync_copy(acc, out_hbm.at[c])


BLK = 1024


def _dinv_block(degp_ref):
    deg = degp_ref[0][:, 0:1] + degp_ref[1][:, 0:1] + 1.0
    return lax.rsqrt(deg)


def _mm1_body(x_ref, w_ref, degp_ref, y_ref):
    dinv = _dinv_block(degp_ref)
    xw = jnp.dot(x_ref[...], w_ref[...], preferred_element_type=jnp.float32)
    y_ref[...] = xw * dinv


def _mid_body(aggp_ref, y1_ref, degp_ref, b1_ref, w2_ref, y2_ref):
    dinv = _dinv_block(degp_ref)
    agg = aggp_ref[0] + aggp_ref[1] + y1_ref[...]
    h = jnp.maximum(agg * dinv + b1_ref[...], 0.0)
    y2_ref[...] = jnp.dot(h, w2_ref[...],
                          preferred_element_type=jnp.float32) * dinv


def _fin_body(aggp_ref, y2_ref, degp_ref, b2_ref, h2_ref):
    dinv = _dinv_block(degp_ref)
    agg = aggp_ref[0] + aggp_ref[1] + y2_ref[...]
    h2_ref[...] = jnp.maximum(agg * dinv + b2_ref[...], 0.0)


def _head_body(poolp_ref, cntp_ref, w1_ref, b1_ref, w2_ref, b2_ref, out_ref):
    p = poolp_ref[0, :G, :] + poolp_ref[1, :G, :]
    cnt = cntp_ref[0, :G, 0:1] + cntp_ref[1, :G, 0:1]
    pooled = p / jnp.maximum(cnt, 1.0)
    c1 = jnp.maximum(
        jnp.dot(pooled, w1_ref[...], preferred_element_type=jnp.float32)
        + b1_ref[...], 0.0)
    out_ref[...] = jnp.dot(c1, w2_ref[...],
                           preferred_element_type=jnp.float32) + b2_ref[...]


def _row_spec(i):
    return (i, 0)


def _full_spec(i):
    return (0, 0)


def _deg_spec(i):
    return (0, i, 0)


def _mm1(x, Wg1, degp):
    return pl.pallas_call(
        _mm1_body,
        grid=(N_PAD // BLK,),
        in_specs=[
            pl.BlockSpec((BLK, D), _row_spec),
            pl.BlockSpec((D, D), _full_spec),
            pl.BlockSpec((2, BLK, D), _deg_spec),
        ],
        out_specs=pl.BlockSpec((BLK, D), _row_spec),
        out_shape=jax.ShapeDtypeStruct((N_PAD, D), jnp.float32),
    )(x, Wg1, degp)


def _mid(aggp, y1, degp, b1, Wg2):
    return pl.pallas_call(
        _mid_body,
        grid=(N_PAD // BLK,),
        in_specs=[
            pl.BlockSpec((2, BLK, D), _deg_spec),
            pl.BlockSpec((BLK, D), _row_spec),
            pl.BlockSpec((2, BLK, D), _deg_spec),
            pl.BlockSpec((1, D), _full_spec),
            pl.BlockSpec((D, D), _full_spec),
        ],
        out_specs=pl.BlockSpec((BLK, D), _row_spec),
        out_shape=jax.ShapeDtypeStruct((N_PAD, D), jnp.float32),
    )(aggp, y1, degp, b1, Wg2)


def _fin(aggp, y2, degp, b2):
    return pl.pallas_call(
        _fin_body,
        grid=(N_PAD // BLK,),
        in_specs=[
            pl.BlockSpec((2, BLK, D), _deg_spec),
            pl.BlockSpec((BLK, D), _row_spec),
            pl.BlockSpec((2, BLK, D), _deg_spec),
            pl.BlockSpec((1, D), _full_spec),
        ],
        out_specs=pl.BlockSpec((BLK, D), _row_spec),
        out_shape=jax.ShapeDtypeStruct((N_PAD, D), jnp.float32),
    )(aggp, y2, degp, b2)


def _head(poolp, cntp, Wc1p, bc1p, Wc2p, bc2p):
    return pl.pallas_call(
        _head_body,
        in_specs=[
            pl.BlockSpec((2, G_PAD, D), lambda: (0, 0, 0)),
            pl.BlockSpec((2, G_PAD, D), lambda: (0, 0, 0)),
            pl.BlockSpec((D, D), lambda: (0, 0)),
            pl.BlockSpec((1, D), lambda: (0, 0)),
            pl.BlockSpec((D, D), lambda: (0, 0)),
            pl.BlockSpec((1, D), lambda: (0, 0)),
        ],
        out_specs=pl.BlockSpec((G, D), lambda: (0, 0)),
        out_shape=jax.ShapeDtypeStruct((G, D), jnp.float32),
    )(poolp, cntp, Wc1p, bc1p, Wc2p, bc2p)


def kernel(x, edge_index, batch, Wg1, bg1, Wg2, bg2, Wc1, bc1, Wc2, bc2):
    f32 = jnp.float32
    x_p = jnp.zeros((N_PAD, D), f32).at[:N].set(x)
    pad_e = E_CHUNKS * CHUNK - E
    src_p = jnp.concatenate(
        [edge_index[0], jnp.zeros((pad_e,), jnp.int32)]).reshape(
            32, EC_TILE, CHUNK)
    dst_p = jnp.concatenate(
        [edge_index[1], jnp.full((pad_e,), DUMMY, jnp.int32)]).reshape(
            32, EC_TILE, CHUNK)
    batch_p = jnp.concatenate(
        [batch, jnp.full((N_PAD - N,), G, jnp.int32)]).reshape(
            P_TILES, BCH_TILE, CHUNK)
    batdst_p = jnp.concatenate(
        [batch, jnp.full((E_CHUNKS * CHUNK - N,), DUMMY, jnp.int32)]).reshape(
            32, EC_TILE, CHUNK)
    zeros128 = jnp.zeros((N_PAD, D), f32)
    ones128 = jnp.ones((N_PAD, D), f32)
    zero_idx = jnp.zeros((32, EC_TILE, CHUNK), jnp.int32)

    degp = _agg(ones128, zero_idx, dst_p, zeros128)
    cntp = _agg(ones128, zero_idx, batdst_p, zeros128)[:, :G_PAD]

    y1 = _mm1(x_p, Wg1, degp)
    agg1 = _agg(y1, src_p, dst_p, zeros128)
    y2 = _mid(agg1, y1, degp, bg1.reshape(1, D), Wg2)
    agg2 = _agg(y2, src_p, dst_p, zeros128)
    h2 = _fin(agg2, y2, degp, bg2.reshape(1, D))
    poolp = _pool(h2, batch_p, zeros128)

    Wc1p = jnp.zeros((D, D), f32).at[:, :64].set(Wc1)
    bc1p = jnp.zeros((1, D), f32).at[0, :64].set(bc1)
    Wc2p = jnp.zeros((D, D), f32).at[:64, 0:1].set(Wc2)
    bc2p = jnp.zeros((1, D), f32).at[0, 0].set(bc2[0])
    out = _head(poolp, cntp, Wc1p, bc1p, Wc2p, bc2p)
    return out[:, 0:1]
